# baseline (speedup 1.0000x reference)
import functools

import jax
import jax.numpy as jnp
from jax.experimental import pallas as pl
from jax.experimental.pallas import tpu as pltpu


def _make_body(c_in, c_out, tile_h, chunk_h=16):
    def body(x_ref, w_ref, b_ref, o_ref):
        for h0 in range(0, tile_h, chunk_h):
            hs = pl.ds(h0, chunk_h)
            accs = None
            for c in range(c_in):
                xv = x_ref[c, hs, :]
                if accs is None:
                    accs = [xv * w_ref[o, c] + b_ref[o] for o in range(c_out)]
                else:
                    accs = [accs[o] + xv * w_ref[o, c] for o in range(c_out)]
            for o in range(c_out):
                o_ref[o, hs, :] = accs[o].astype(o_ref.dtype)
    return body


@functools.partial(jax.jit, static_argnames=("tile_h",))
def _outconv4d(x, w, b, *, tile_h=128):
    N, C_in, H, W = x.shape
    C_out = w.shape[0]

    w2 = w.reshape(C_out, C_in).astype(jnp.float32)
    b1 = b.astype(jnp.float32)

    th = H if H <= tile_h else tile_h
    num_h = pl.cdiv(H, th)
    grid = (N, num_h)

    x_bytes = 2 * C_in * th * W * x.dtype.itemsize
    o_bytes = 2 * C_out * th * W * x.dtype.itemsize
    vmem_limit = int(min(x_bytes + o_bytes + (8 << 20), 56 << 20))

    out = pl.pallas_call(
        _make_body(C_in, C_out, th),
        out_shape=jax.ShapeDtypeStruct((N, C_out, H, W), x.dtype),
        grid=grid,
        in_specs=[
            pl.BlockSpec((None, C_in, th, W), lambda n, h: (n, 0, h, 0)),
            pl.BlockSpec(memory_space=pltpu.MemorySpace.SMEM),
            pl.BlockSpec(memory_space=pltpu.MemorySpace.SMEM),
        ],
        out_specs=pl.BlockSpec((None, C_out, th, W), lambda n, h: (n, 0, h, 0)),
        compiler_params=pltpu.CompilerParams(
            dimension_semantics=("parallel", "parallel"),
            vmem_limit_bytes=vmem_limit,
        ),
    )(x, w2, b1)

    return out


def kernel(x, w, b):
    return _outconv4d(x, w, b)
